# baseline (device time: 40067 ns/iter reference)
import jax
import jax.numpy as jnp
from jax import lax
from jax.experimental import pallas as pl
from jax.experimental.pallas import tpu as pltpu

N_DEV = 8

_MASKS = {
    "A": (3, 1, 4),
    "B": (1, 4, 3),
    "C": (4, 3, 1),
}
_SPLITS = {
    "A": (lambda p: (p >> 1) & 1, lambda p: p & 1, lambda p: (p >> 2) & 1),
    "B": (lambda p: (p ^ (p >> 1)) & 1, lambda p: (p >> 2) & 1, lambda p: (p >> 1) & 1),
    "C": (lambda p: (p >> 2) & 1, lambda p: (p >> 1) & 1, lambda p: p & 1),
}

_CHAINS = (
    (0, 384, "A"),
    (704, 384, "B"),
    (1408, 320, "C"),
    (384, 320, "A"),
    (1088, 320, "B"),
    (1728, 320, "C"),
)


def kernel(x):
    m, n = x.shape
    nc = len(_CHAINS)

    def body(x_ref, out_ref, *scratch):
        recv_bufs = scratch[:nc]
        send_sems, recv_sems = scratch[nc], scratch[nc + 1]
        p = lax.axis_index("i")

        chains = []
        for base, rows, part in _CHAINS:
            chains.append(
                {"base": base, "rows": rows, "masks": _MASKS[part],
                 "s": [fn(p) for fn in _SPLITS[part]]}
            )

        barrier_sem = pltpu.get_barrier_semaphore()
        for mask in (1, 3, 4):
            pl.semaphore_signal(
                barrier_sem, inc=1,
                device_id=(p ^ mask,), device_id_type=pl.DeviceIdType.MESH,
            )
        pl.semaphore_wait(barrier_sem, 3)

        cur = [ch["base"] for ch in chains]
        rdmas = [[None] * 6 for _ in chains]

        def start_stage(ci, s):
            ch = chains[ci]
            if s < 3:
                r = s
                half = ch["rows"] >> (r + 1)
                roff = ch["rows"] - (ch["rows"] >> r)
                sp = ch["s"][r]
                rdma = pltpu.make_async_remote_copy(
                    src_ref=out_ref.at[pl.ds(cur[ci] + (1 - sp) * half, half), :],
                    dst_ref=recv_bufs[ci].at[pl.ds(roff, half), :],
                    send_sem=send_sems.at[ci * 6 + s],
                    recv_sem=recv_sems.at[ci * 6 + s],
                    device_id=(p ^ ch["masks"][r],),
                    device_id_type=pl.DeviceIdType.MESH,
                )
            else:
                r = 5 - s
                half = ch["rows"] >> (r + 1)
                rdma = pltpu.make_async_remote_copy(
                    src_ref=out_ref.at[pl.ds(cur[ci], half), :],
                    dst_ref=out_ref.at[pl.ds(cur[ci], half), :],
                    send_sem=send_sems.at[ci * 6 + s],
                    recv_sem=recv_sems.at[ci * 6 + s],
                    device_id=(p ^ ch["masks"][r],),
                    device_id_type=pl.DeviceIdType.MESH,
                )
            rdma.start()
            rdmas[ci][s] = rdma

        def finish_stage(ci, s):
            ch = chains[ci]
            rdmas[ci][s].wait()
            if s < 3:
                r = s
                half = ch["rows"] >> (r + 1)
                roff = ch["rows"] - (ch["rows"] >> r)
                cur[ci] = cur[ci] + ch["s"][r] * half
                out_ref[pl.ds(cur[ci], half), :] += recv_bufs[ci][
                    pl.ds(roff, half), :
                ]
            else:
                r = 5 - s
                half = ch["rows"] >> (r + 1)
                cur[ci] = cur[ci] - ch["s"][r] * half

        for ci, ch in enumerate(chains):
            half = ch["rows"] >> 1
            soff = ch["base"] + (1 - ch["s"][0]) * half
            out_ref[pl.ds(soff, half), :] = x_ref[
                pl.ds(soff, half), :
            ].astype(jnp.bfloat16)
            start_stage(ci, 0)
        for ci, ch in enumerate(chains):
            half = ch["rows"] >> 1
            koff = ch["base"] + ch["s"][0] * half
            out_ref[pl.ds(koff, half), :] = x_ref[
                pl.ds(koff, half), :
            ].astype(jnp.bfloat16)

        for s in range(6):
            for ci in range(nc):
                finish_stage(ci, s)
                if s < 5:
                    start_stage(ci, s + 1)

    return pl.pallas_call(
        body,
        out_shape=jax.ShapeDtypeStruct((m, n), jnp.bfloat16),
        in_specs=[pl.BlockSpec(memory_space=pltpu.VMEM)],
        out_specs=pl.BlockSpec(memory_space=pltpu.VMEM),
        scratch_shapes=[
            pltpu.VMEM((rows * 7 // 8, n), jnp.bfloat16)
            for _, rows, _ in _CHAINS
        ] + [
            pltpu.SemaphoreType.DMA((nc * 6,)),
            pltpu.SemaphoreType.DMA((nc * 6,)),
        ],
        compiler_params=pltpu.CompilerParams(collective_id=0),
    )(x)


# device time: 38320 ns/iter; 1.0456x vs baseline; 1.0456x over previous
import jax
import jax.numpy as jnp
from jax import lax
from jax.experimental import pallas as pl
from jax.experimental.pallas import tpu as pltpu

N_DEV = 8

_MASKS = {
    "A": (3, 1, 4),
    "B": (1, 4, 3),
    "C": (4, 3, 1),
}
_SPLITS = {
    "A": (lambda p: (p >> 1) & 1, lambda p: p & 1, lambda p: (p >> 2) & 1),
    "B": (lambda p: (p ^ (p >> 1)) & 1, lambda p: (p >> 2) & 1, lambda p: (p >> 1) & 1),
    "C": (lambda p: (p >> 2) & 1, lambda p: (p >> 1) & 1, lambda p: p & 1),
}

_CHAINS = (
    (0, 192, "A"), (704, 192, "B"), (1408, 192, "C"),
    (192, 192, "A"), (896, 192, "B"), (1600, 192, "C"),
    (384, 192, "A"), (1088, 192, "B"), (1792, 128, "C"),
    (576, 128, "A"), (1280, 128, "B"), (1920, 128, "C"),
)

_N_STAGES = 5


def kernel(x):
    m, n = x.shape
    nc = len(_CHAINS)

    def body(x_ref, out_ref, *scratch):
        recv_bufs = scratch[:nc]
        send_sems, recv_sems = scratch[nc], scratch[nc + 1]
        p = lax.axis_index("i")

        chains = []
        for base, rows, part in _CHAINS:
            chains.append(
                {"base": base, "rows": rows, "masks": _MASKS[part],
                 "s": [fn(p) for fn in _SPLITS[part]]}
            )

        barrier_sem = pltpu.get_barrier_semaphore()
        for mask in (1, 3, 4):
            pl.semaphore_signal(
                barrier_sem, inc=1,
                device_id=(p ^ mask,), device_id_type=pl.DeviceIdType.MESH,
            )
        pl.semaphore_wait(barrier_sem, 3)

        cur = [ch["base"] for ch in chains]
        rdmas = [[None] * _N_STAGES for _ in chains]

        def start_stage(ci, s):
            ch = chains[ci]
            R = ch["rows"]
            if s < 2:
                half = R >> (s + 1)
                roff = (0, R >> 1)[s]
                sp = ch["s"][s]
                src = out_ref.at[pl.ds(cur[ci] + (1 - sp) * half, half), :]
                dst = recv_bufs[ci].at[pl.ds(roff, half), :]
            elif s == 2:
                q = R >> 2
                src = out_ref.at[pl.ds(cur[ci], q), :]
                dst = recv_bufs[ci].at[pl.ds(3 * (R >> 2), q), :]
            else:
                half = R >> (5 - s)
                src = out_ref.at[pl.ds(cur[ci], half), :]
                dst = out_ref.at[pl.ds(cur[ci], half), :]
            mask = ch["masks"][s if s < 3 else 4 - s]
            rdma = pltpu.make_async_remote_copy(
                src_ref=src,
                dst_ref=dst,
                send_sem=send_sems.at[ci * _N_STAGES + s],
                recv_sem=recv_sems.at[ci * _N_STAGES + s],
                device_id=(p ^ mask,),
                device_id_type=pl.DeviceIdType.MESH,
            )
            rdma.start()
            rdmas[ci][s] = rdma

        def finish_stage(ci, s):
            ch = chains[ci]
            R = ch["rows"]
            rdmas[ci][s].wait()
            if s < 2:
                half = R >> (s + 1)
                roff = (0, R >> 1)[s]
                cur[ci] = cur[ci] + ch["s"][s] * half
                out_ref[pl.ds(cur[ci], half), :] += recv_bufs[ci][
                    pl.ds(roff, half), :
                ]
            elif s == 2:
                q = R >> 2
                out_ref[pl.ds(cur[ci], q), :] += recv_bufs[ci][
                    pl.ds(3 * (R >> 2), q), :
                ]
            else:
                half = R >> (5 - s)
                cur[ci] = cur[ci] - ch["s"][4 - s] * half

        for ci, ch in enumerate(chains):
            half = ch["rows"] >> 1
            soff = ch["base"] + (1 - ch["s"][0]) * half
            out_ref[pl.ds(soff, half), :] = x_ref[
                pl.ds(soff, half), :
            ].astype(jnp.bfloat16)
            start_stage(ci, 0)
        for ci, ch in enumerate(chains):
            half = ch["rows"] >> 1
            koff = ch["base"] + ch["s"][0] * half
            out_ref[pl.ds(koff, half), :] = x_ref[
                pl.ds(koff, half), :
            ].astype(jnp.bfloat16)

        for s in range(_N_STAGES):
            for ci in range(nc):
                finish_stage(ci, s)
                if s < _N_STAGES - 1:
                    start_stage(ci, s + 1)

    return pl.pallas_call(
        body,
        out_shape=jax.ShapeDtypeStruct((m, n), jnp.bfloat16),
        in_specs=[pl.BlockSpec(memory_space=pltpu.VMEM)],
        out_specs=pl.BlockSpec(memory_space=pltpu.VMEM),
        scratch_shapes=[
            pltpu.VMEM((rows, n), jnp.bfloat16) for _, rows, _ in _CHAINS
        ] + [
            pltpu.SemaphoreType.DMA((nc * _N_STAGES,)),
            pltpu.SemaphoreType.DMA((nc * _N_STAGES,)),
        ],
        compiler_params=pltpu.CompilerParams(collective_id=0),
    )(x)
